# Initial kernel scaffold; baseline (speedup 1.0000x reference)
import jax
import jax.numpy as jnp
from jax.experimental import pallas as pl


def kernel(y_true, y_soft, spread_idx, eval_idx, test_idx, edge_index):
    raise NotImplementedError("write your pallas kernel here")


# speedup vs baseline: 3.1472x; 3.1472x over previous
from functools import partial

import jax
import jax.numpy as jnp
from jax import lax
from jax.experimental import pallas as pl
from jax.experimental.pallas import tpu as pltpu, tpu_sc as plsc

N = 100000
E = 1600000
C = 40
HALF = N // 2
NDUMP = 16
ACC_ROWS = HALF + NDUMP
B = 40
EPW = E // 16
NPAIR = EPW // (2 * B)
ROWS_PW = HALF // 16
ZCH = 248
SPAD = 51200
SPW = SPAD // 16
DRA = 3128
DRB = 50000 - 15 * DRA
ZRA = 3128
ZRB = ACC_ROWS - 15 * ZRA

_mesh = plsc.VectorSubcoreMesh(core_axis_name="c", subcore_axis_name="s")


def _z16():
    return jnp.zeros((16,), jnp.float32)


def _agg_body(xs_hbm, src_hbm, dstl_hbm, agg_hbm,
              acc, sidx0, sidx1, didx0, didx1, rows0, rows1,
              sem0, sem1):
    c = lax.axis_index("c")
    s = lax.axis_index("s")

    def zrow(i, _):
        z = _z16()
        rows0[i, pl.ds(0, 16)] = z
        rows0[i, pl.ds(16, 16)] = z
        rows0[i, pl.ds(24, 16)] = z
        return 0
    lax.fori_loop(0, B, zrow, 0)
    zbase = s * (ACC_ROWS // 16)

    def zcopy(k, _):
        pltpu.sync_copy(rows0, acc.at[pl.ds(zbase + k * B, B)])
        return 0
    lax.fori_loop(0, 3126 // B, zcopy, 0)
    pltpu.sync_copy(rows0.at[pl.ds(0, 3126 - (3126 // B) * B)],
                    acc.at[pl.ds(zbase + (3126 // B) * B,
                                 3126 - (3126 // B) * B)])
    plsc.subcore_barrier()

    ebase = s * EPW

    def issue(j, sidx, didx, rows, sem):
        off = ebase + j * B
        pltpu.sync_copy(src_hbm.at[pl.ds(off, B)], sidx)
        pltpu.sync_copy(dstl_hbm.at[pl.ds(c * E + off, B)], didx)
        pltpu.async_copy(xs_hbm.at[sidx], rows, sem)

    def drain(sidx, rows, sem, didx):
        pltpu.make_async_copy(xs_hbm.at[sidx], rows, sem).wait()
        pltpu.sync_copy(rows, acc.at[didx], add=True)

    issue(0, sidx0, didx0, rows0, sem0)
    issue(1, sidx1, didx1, rows1, sem1)

    def body(i, _):
        drain(sidx0, rows0, sem0, didx0)
        issue(2 * i + 2, sidx0, didx0, rows0, sem0)
        drain(sidx1, rows1, sem1, didx1)
        issue(2 * i + 3, sidx1, didx1, rows1, sem1)
        return 0
    lax.fori_loop(0, NPAIR - 1, body, 0)
    drain(sidx0, rows0, sem0, didx0)
    drain(sidx1, rows1, sem1, didx1)

    plsc.subcore_barrier()

    def dchunk(k, _):
        base = s * DRA + k * B
        pltpu.sync_copy(acc.at[pl.ds(base, B)], rows0)
        pltpu.sync_copy(rows0, agg_hbm.at[pl.ds(c * HALF + base, B)])
        return 0

    @pl.when(s < 15)
    def _():
        lax.fori_loop(0, 78, dchunk, 0)
        base = s * DRA + 78 * B
        pltpu.sync_copy(acc.at[pl.ds(base, 8)], rows1.at[pl.ds(0, 8)])
        pltpu.sync_copy(rows1.at[pl.ds(0, 8)],
                        agg_hbm.at[pl.ds(c * HALF + base, 8)])

    @pl.when(s == 15)
    def _():
        lax.fori_loop(0, 77, dchunk, 0)


_agg_call = pl.kernel(
    _agg_body,
    out_type=jax.ShapeDtypeStruct((N, C), jnp.float32),
    mesh=_mesh,
    scratch_types=[
        pltpu.VMEM_SHARED((ACC_ROWS, C), jnp.float32),
        pltpu.VMEM((B,), jnp.int32),
        pltpu.VMEM((B,), jnp.int32),
        pltpu.VMEM((B,), jnp.int32),
        pltpu.VMEM((B,), jnp.int32),
        pltpu.VMEM((B, C), jnp.float32),
        pltpu.VMEM((B, C), jnp.float32),
        pltpu.SemaphoreType.DMA,
        pltpu.SemaphoreType.DMA,
    ],
    compiler_params=pltpu.CompilerParams(use_tc_tiling_on_sc=False),
)


def _hist_body(dstl_hbm, sprl_hbm, deg_hbm, cnt_hbm,
               acc_d, acc_c, ones, zrow, idxb):
    c = lax.axis_index("c")
    s = lax.axis_index("s")

    def fill(i, _):
        ones[pl.ds(i * 16, 16)] = _z16() + 1.0
        zrow[pl.ds(i * 16, 16)] = _z16()
        return 0
    lax.fori_loop(0, 16, fill, 0)

    def zseg(i, _):
        zrow[pl.ds(i * 16 + 256, 16)] = _z16()
        return 0
    lax.fori_loop(0, (3200 - 256) // 16, zseg, 0)

    @pl.when(s < 15)
    def _():
        pltpu.sync_copy(zrow.at[pl.ds(0, ZRA)], acc_d.at[pl.ds(s * ZRA, ZRA)])
        pltpu.sync_copy(zrow.at[pl.ds(0, ZRA)], acc_c.at[pl.ds(s * ZRA, ZRA)])

    @pl.when(s == 15)
    def _():
        pltpu.sync_copy(zrow.at[pl.ds(0, ZRB)], acc_d.at[pl.ds(15 * ZRA, ZRB)])
        pltpu.sync_copy(zrow.at[pl.ds(0, ZRB)], acc_c.at[pl.ds(15 * ZRA, ZRB)])
    plsc.subcore_barrier()

    ebase = s * EPW

    def dbody(j, _):
        pltpu.sync_copy(dstl_hbm.at[pl.ds(c * E + ebase + j * B, B)], idxb)
        pltpu.sync_copy(ones.at[pl.ds(0, B)], acc_d.at[idxb], add=True)
        return 0
    lax.fori_loop(0, EPW // B, dbody, 0)

    sbase = s * SPW

    def sbody(j, _):
        pltpu.sync_copy(sprl_hbm.at[pl.ds(c * SPAD + sbase + j * B, B)], idxb)
        pltpu.sync_copy(ones.at[pl.ds(0, B)], acc_c.at[idxb], add=True)
        return 0
    lax.fori_loop(0, SPW // B, sbody, 0)

    plsc.subcore_barrier()

    @pl.when(s < 15)
    def _():
        pltpu.sync_copy(acc_d.at[pl.ds(s * DRA, DRA)], zrow.at[pl.ds(0, DRA)])
        pltpu.sync_copy(zrow.at[pl.ds(0, DRA)],
                        deg_hbm.at[pl.ds(c * HALF + s * DRA, DRA)])
        pltpu.sync_copy(acc_c.at[pl.ds(s * DRA, DRA)], zrow.at[pl.ds(0, DRA)])
        pltpu.sync_copy(zrow.at[pl.ds(0, DRA)],
                        cnt_hbm.at[pl.ds(c * HALF + s * DRA, DRA)])

    @pl.when(s == 15)
    def _():
        pltpu.sync_copy(acc_d.at[pl.ds(15 * DRA, DRB)], zrow.at[pl.ds(0, DRB)])
        pltpu.sync_copy(zrow.at[pl.ds(0, DRB)],
                        deg_hbm.at[pl.ds(c * HALF + 15 * DRA, DRB)])
        pltpu.sync_copy(acc_c.at[pl.ds(15 * DRA, DRB)], zrow.at[pl.ds(0, DRB)])
        pltpu.sync_copy(zrow.at[pl.ds(0, DRB)],
                        cnt_hbm.at[pl.ds(c * HALF + 15 * DRA, DRB)])


_hist_call = pl.kernel(
    _hist_body,
    out_type=(jax.ShapeDtypeStruct((N,), jnp.float32),
              jax.ShapeDtypeStruct((N,), jnp.float32)),
    mesh=_mesh,
    scratch_types=[
        pltpu.VMEM_SHARED((ACC_ROWS,), jnp.float32),
        pltpu.VMEM_SHARED((ACC_ROWS,), jnp.float32),
        pltpu.VMEM((256,), jnp.float32),
        pltpu.VMEM((3200,), jnp.float32),
        pltpu.VMEM((B,), jnp.int32),
    ],
)


BM = 1000
GRID = N // BM


def _prep_tc(y_true_ref, y_soft_ref, deg_ref, cnt_ref,
             dis_ref, e0_ref, e0s_ref, sigma_ref):
    i = pl.program_id(0)
    deg = deg_ref[...]
    cnt = cnt_ref[...]
    dis = jnp.where(deg > 0, lax.rsqrt(jnp.maximum(deg, 1.0)), 0.0)
    dis_ref[...] = dis
    yt = y_true_ref[...]
    oh = (lax.broadcasted_iota(jnp.int32, (BM, C), 1) == yt).astype(jnp.float32)
    e0 = jnp.where(cnt > 0, oh - y_soft_ref[...], 0.0)
    e0_ref[...] = e0
    e0s_ref[...] = dis * e0
    part = jnp.sum(cnt * jnp.sum(jnp.abs(e0), axis=1, keepdims=True))

    @pl.when(i == 0)
    def _():
        sigma_ref[...] = jnp.zeros_like(sigma_ref)
    sigma_ref[...] = sigma_ref[...] + part


def _post_tc(agg_ref, y0_ref, dis_ref, x_ref, xs_ref, *, alpha, mode):
    dis = dis_ref[...]
    x = alpha * dis * agg_ref[...] + (1.0 - alpha) * y0_ref[...]
    if mode == "clip":
        x = jnp.clip(x, -1.0, 1.0)
    else:
        x = x - jnp.max(x, axis=1, keepdims=True)
        ex = jnp.exp(x)
        x = ex / jnp.sum(ex, axis=1, keepdims=True)
    x_ref[...] = x
    xs_ref[...] = dis * x


def _mid_tc(x10_ref, y_soft_ref, y_true_ref, cnt_ref, dis_ref, sigma_ref,
            y_ref, ys_ref):
    x10 = x10_ref[...]
    sigma = sigma_ref[...][0, 0] * (1.0 / 50000.0)
    denom = jnp.sum(jnp.abs(x10), axis=1, keepdims=True)
    bad = (denom <= 1e-12) | ((sigma / jnp.maximum(denom, 1e-12)) > 1000.0)
    scale = jnp.where(bad, 1.0, sigma / jnp.where(bad, 1.0, denom))
    y_corr = y_soft_ref[...] + scale * x10
    yt = y_true_ref[...]
    oh = (lax.broadcasted_iota(jnp.int32, (BM, C), 1) == yt).astype(jnp.float32)
    y = jnp.where(cnt_ref[...] > 0, oh, y_corr)
    y_ref[...] = y
    ys_ref[...] = dis_ref[...] * y


def _row_spec(w):
    return pl.BlockSpec((BM, w), lambda i: (i, 0))


def _one_spec():
    return pl.BlockSpec((1, 1), lambda i: (0, 0))


_prep_call = pl.pallas_call(
    _prep_tc,
    grid=(GRID,),
    in_specs=[_row_spec(1), _row_spec(C), _row_spec(1), _row_spec(1)],
    out_specs=[_row_spec(1), _row_spec(C), _row_spec(C), _one_spec()],
    out_shape=[jax.ShapeDtypeStruct((N, 1), jnp.float32),
               jax.ShapeDtypeStruct((N, C), jnp.float32),
               jax.ShapeDtypeStruct((N, C), jnp.float32),
               jax.ShapeDtypeStruct((1, 1), jnp.float32)],
)


def _post_call(alpha, mode):
    return pl.pallas_call(
        partial(_post_tc, alpha=alpha, mode=mode),
        grid=(GRID,),
        in_specs=[_row_spec(C), _row_spec(C), _row_spec(1)],
        out_specs=[_row_spec(C), _row_spec(C)],
        out_shape=[jax.ShapeDtypeStruct((N, C), jnp.float32),
                   jax.ShapeDtypeStruct((N, C), jnp.float32)],
    )


_mid_call = pl.pallas_call(
    _mid_tc,
    grid=(GRID,),
    in_specs=[_row_spec(C), _row_spec(C), _row_spec(1), _row_spec(1),
              _row_spec(1), _one_spec()],
    out_specs=[_row_spec(C), _row_spec(C)],
    out_shape=[jax.ShapeDtypeStruct((N, C), jnp.float32),
               jax.ShapeDtypeStruct((N, C), jnp.float32)],
)

_post_c = _post_call(0.9, "clip")
_post_s = _post_call(0.8, "softmax")


def kernel(y_true, y_soft, spread_idx, eval_idx, test_idx, edge_index):
    src = edge_index[0].astype(jnp.int32)
    dst = edge_index[1].astype(jnp.int32)
    spread = spread_idx.astype(jnp.int32)

    e_iota = jnp.arange(E, dtype=jnp.int32)
    dump = HALF + (e_iota & (NDUMP - 1))
    dstl = jnp.concatenate([jnp.where(dst < HALF, dst, dump),
                            jnp.where(dst >= HALF, dst - HALF, dump)])
    s_iota = jnp.arange(SPAD, dtype=jnp.int32)
    sdump = HALF + (s_iota & (NDUMP - 1))
    spad = jnp.concatenate([spread, jnp.full((SPAD - 50000,), -1, jnp.int32)])
    sprl = jnp.concatenate([
        jnp.where((spad >= 0) & (spad < HALF), spad, sdump),
        jnp.where(spad >= HALF, spad - HALF, sdump)])

    deg, cnt = _hist_call(dstl, sprl)

    yt2 = y_true.astype(jnp.int32).reshape(N, 1)
    deg2 = deg.reshape(N, 1)
    cnt2 = cnt.reshape(N, 1)
    dis, e0, e0s, sigma = _prep_call(yt2, y_soft, deg2, cnt2)

    x, xs = e0, e0s
    for _ in range(10):
        agg = _agg_call(xs, src, dstl)
        x, xs = _post_c(agg, e0, dis)

    y, ys = _mid_call(x, y_soft, yt2, cnt2, dis, sigma)

    x, xs = y, ys
    for _ in range(10):
        agg = _agg_call(xs, src, dstl)
        x, xs = _post_s(agg, y, dis)

    return x


# speedup vs baseline: 7.6918x; 2.4440x over previous
from functools import partial

import jax
import jax.numpy as jnp
from jax import lax
from jax.experimental import pallas as pl
from jax.experimental.pallas import tpu as pltpu, tpu_sc as plsc

N = 100000
E = 1600000
C = 40
CH = C // 2
CP = 24
HALF = N // 2
NDUMP = 16
ACC_ROWS = HALF + NDUMP
B = 128
K = 8
IT = 100
SLOTS = IT * K * B
ETOT = 16 * SLOTS
RPW = IT * K
DRA = 3128
DRB = 50000 - 15 * DRA
SPAD = 51200
SSPW = SPAD // 16

_mesh = plsc.VectorSubcoreMesh(core_axis_name="c", subcore_axis_name="s")


def _z16():
    return jnp.zeros((16,), jnp.float32)


def _agg_body(xsa_hbm, xsb_hbm, srcp_hbm, dstl_hbm, agga_hbm, aggb_hbm,
              acc, dbuf, *scr):
    sidx = scr[0:8]
    didx = scr[8:16]
    rows = scr[16:24]
    gsem, isem_s, isem_d = scr[24:27]
    c = lax.axis_index("c")
    s = lax.axis_index("s")

    def zero_dbuf():
        def zr(i, _):
            z = _z16()
            dbuf[i, pl.ds(0, 16)] = z
            dbuf[i, pl.ds(8, 16)] = z
            return 0
        lax.fori_loop(0, 1024, zr, 0)

    def zero_acc():
        zbase = s * (ACC_ROWS // 16)

        def zc(k, _):
            pltpu.sync_copy(dbuf, acc.at[pl.ds(zbase + k * 1024, 1024)])
            return 0
        lax.fori_loop(0, 3, zc, 0)
        pltpu.sync_copy(dbuf.at[pl.ds(0, 54)],
                        acc.at[pl.ds(zbase + 3072, 54)])

    sbase = s * SLOTS
    dbase0 = s * SLOTS
    zero_dbuf()

    def run_pass(xs_hbm, agg_hbm):
        zero_acc()
        plsc.subcore_barrier()

        def i_issue(j, b):
            off = sbase + j * B
            doff = c * ETOT + dbase0 + j * B
            pltpu.async_copy(srcp_hbm.at[pl.ds(off, B)], sidx[b],
                             isem_s.at[b])
            pltpu.async_copy(dstl_hbm.at[pl.ds(doff, B)], didx[b],
                             isem_d.at[b])

        def i_wait(b):
            pltpu.make_async_copy(srcp_hbm.at[pl.ds(sbase, B)], sidx[b],
                                  isem_s.at[b]).wait()
            pltpu.make_async_copy(dstl_hbm.at[pl.ds(sbase, B)], didx[b],
                                  isem_d.at[b]).wait()

        def g_issue(b):
            pltpu.async_copy(xs_hbm.at[sidx[b]], rows[b], gsem.at[b])

        def g_wait(b):
            pltpu.make_async_copy(xs_hbm.at[sidx[b]], rows[b],
                                  gsem.at[b]).wait()

        for b in range(8):
            i_issue(b, b)

        def body(i, _):
            for h in range(2):
                sl = range(4 * h, 4 * h + 4)
                for b in sl:
                    i_wait(b)
                    g_issue(b)
                for b in sl:
                    g_wait(b)
                    pltpu.sync_copy(rows[b], acc.at[didx[b]], add=True)
                    i_issue(jnp.minimum(8 * i + 8 + b, SLOTS // B - 1), b)
            return 0
        lax.fori_loop(0, IT, body, 0)
        for b in range(8):
            i_wait(b)

        plsc.subcore_barrier()

        def dchunk(base, sz):
            pltpu.sync_copy(acc.at[pl.ds(base, sz)], dbuf.at[pl.ds(0, sz)])
            pltpu.sync_copy(dbuf.at[pl.ds(0, sz)],
                            agg_hbm.at[pl.ds(c * HALF + base, sz)])

        @pl.when(s < 15)
        def _():
            def dc(k, _):
                base = s * DRA + k * 1024
                pltpu.sync_copy(acc.at[pl.ds(base, 1024)], dbuf)
                pltpu.sync_copy(dbuf,
                                agg_hbm.at[pl.ds(c * HALF + base, 1024)])
                return 0
            lax.fori_loop(0, 3, dc, 0)
            dchunk(s * DRA + 3072, DRA - 3072)

        @pl.when(s == 15)
        def _():
            def dc(k, _):
                base = 15 * DRA + k * 1024
                pltpu.sync_copy(acc.at[pl.ds(base, 1024)], dbuf)
                pltpu.sync_copy(dbuf,
                                agg_hbm.at[pl.ds(c * HALF + base, 1024)])
                return 0
            lax.fori_loop(0, 3, dc, 0)
            dchunk(15 * DRA + 3072, DRB - 3072)
        zero_dbuf()
        plsc.subcore_barrier()

    run_pass(xsa_hbm, agga_hbm)
    run_pass(xsb_hbm, aggb_hbm)


_agg_call = pl.kernel(
    _agg_body,
    out_type=(jax.ShapeDtypeStruct((N, CP), jnp.float32),
              jax.ShapeDtypeStruct((N, CP), jnp.float32)),
    mesh=_mesh,
    scratch_types=(
        [pltpu.VMEM_SHARED((ACC_ROWS, CP), jnp.float32),
         pltpu.VMEM((1024, CP), jnp.float32)]
        + [pltpu.VMEM((B,), jnp.int32) for _ in range(16)]
        + [pltpu.VMEM((B, CP), jnp.float32) for _ in range(8)]
        + [pltpu.SemaphoreType.DMA((8,)) for _ in range(3)]
    ),
    compiler_params=pltpu.CompilerParams(use_tc_tiling_on_sc=False),
)


def _hist_body(dstl_hbm, sprl_hbm, deg_hbm, cnt_hbm,
               acc_d, acc_c, ones, zrow, idxb):
    c = lax.axis_index("c")
    s = lax.axis_index("s")

    def fill(i, _):
        ones[pl.ds(i * 16, 16)] = _z16() + 1.0
        zrow[pl.ds(i * 16, 16)] = _z16()
        return 0
    lax.fori_loop(0, 8, fill, 0)

    def zseg(i, _):
        zrow[pl.ds(i * 16 + 128, 16)] = _z16()
        return 0
    lax.fori_loop(0, (3200 - 128) // 16, zseg, 0)

    ZRA, ZRB = 3128, ACC_ROWS - 15 * 3128

    @pl.when(s < 15)
    def _():
        pltpu.sync_copy(zrow.at[pl.ds(0, ZRA)], acc_d.at[pl.ds(s * ZRA, ZRA)])
        pltpu.sync_copy(zrow.at[pl.ds(0, ZRA)], acc_c.at[pl.ds(s * ZRA, ZRA)])

    @pl.when(s == 15)
    def _():
        pltpu.sync_copy(zrow.at[pl.ds(0, ZRB)], acc_d.at[pl.ds(15 * ZRA, ZRB)])
        pltpu.sync_copy(zrow.at[pl.ds(0, ZRB)], acc_c.at[pl.ds(15 * ZRA, ZRB)])
    plsc.subcore_barrier()

    ebase = s * SLOTS

    def dbody(j, _):
        pltpu.sync_copy(dstl_hbm.at[pl.ds(c * ETOT + ebase + j * B, B)], idxb)
        pltpu.sync_copy(ones.at[pl.ds(0, B)], acc_d.at[idxb], add=True)
        return 0
    lax.fori_loop(0, SLOTS // B, dbody, 0)

    sbase = s * SSPW

    def sbody(j, _):
        pltpu.sync_copy(sprl_hbm.at[pl.ds(c * SPAD + sbase + j * B, B)], idxb)
        pltpu.sync_copy(ones.at[pl.ds(0, B)], acc_c.at[idxb], add=True)
        return 0
    lax.fori_loop(0, SSPW // B, sbody, 0)

    plsc.subcore_barrier()

    @pl.when(s < 15)
    def _():
        pltpu.sync_copy(acc_d.at[pl.ds(s * DRA, DRA)], zrow.at[pl.ds(0, DRA)])
        pltpu.sync_copy(zrow.at[pl.ds(0, DRA)],
                        deg_hbm.at[pl.ds(c * HALF + s * DRA, DRA)])
        pltpu.sync_copy(acc_c.at[pl.ds(s * DRA, DRA)], zrow.at[pl.ds(0, DRA)])
        pltpu.sync_copy(zrow.at[pl.ds(0, DRA)],
                        cnt_hbm.at[pl.ds(c * HALF + s * DRA, DRA)])

    @pl.when(s == 15)
    def _():
        pltpu.sync_copy(acc_d.at[pl.ds(15 * DRA, DRB)], zrow.at[pl.ds(0, DRB)])
        pltpu.sync_copy(zrow.at[pl.ds(0, DRB)],
                        deg_hbm.at[pl.ds(c * HALF + 15 * DRA, DRB)])
        pltpu.sync_copy(acc_c.at[pl.ds(15 * DRA, DRB)], zrow.at[pl.ds(0, DRB)])
        pltpu.sync_copy(zrow.at[pl.ds(0, DRB)],
                        cnt_hbm.at[pl.ds(c * HALF + 15 * DRA, DRB)])


_hist_call = pl.kernel(
    _hist_body,
    out_type=(jax.ShapeDtypeStruct((N,), jnp.float32),
              jax.ShapeDtypeStruct((N,), jnp.float32)),
    mesh=_mesh,
    scratch_types=[
        pltpu.VMEM_SHARED((ACC_ROWS,), jnp.float32),
        pltpu.VMEM_SHARED((ACC_ROWS,), jnp.float32),
        pltpu.VMEM((B,), jnp.float32),
        pltpu.VMEM((3200,), jnp.float32),
        pltpu.VMEM((B,), jnp.int32),
    ],
    compiler_params=pltpu.CompilerParams(use_tc_tiling_on_sc=False),
)


BM = 1000
GRID = N // BM


def _prep_tc(y_true_ref, y_soft_ref, deg_ref, cnt_ref,
             dis_ref, e0_ref, e0sa_ref, e0sb_ref, sigma_ref):
    i = pl.program_id(0)
    deg = deg_ref[...]
    cnt = cnt_ref[...]
    dis = jnp.where(deg > 0, lax.rsqrt(jnp.maximum(deg, 1.0)), 0.0)
    dis_ref[...] = dis
    yt = y_true_ref[...]
    oh = (lax.broadcasted_iota(jnp.int32, (BM, C), 1) == yt).astype(jnp.float32)
    e0 = jnp.where(cnt > 0, oh - y_soft_ref[...], 0.0)
    e0_ref[...] = e0
    e0s = dis * e0
    zpad = jnp.zeros((BM, CP - CH), jnp.float32)
    e0sa_ref[...] = jnp.concatenate([e0s[:, :CH], zpad], axis=1)
    e0sb_ref[...] = jnp.concatenate([e0s[:, CH:], zpad], axis=1)
    part = jnp.sum(cnt * jnp.sum(jnp.abs(e0), axis=1, keepdims=True))

    @pl.when(i == 0)
    def _():
        sigma_ref[...] = jnp.zeros_like(sigma_ref)
    sigma_ref[...] = sigma_ref[...] + part


def _post_tc(agga_ref, aggb_ref, y0_ref, dis_ref, x_ref, xsa_ref, xsb_ref,
             *, alpha, mode):
    dis = dis_ref[...]
    agg = jnp.concatenate([agga_ref[...][:, :CH], aggb_ref[...][:, :CH]],
                          axis=1)
    x = alpha * dis * agg + (1.0 - alpha) * y0_ref[...]
    if mode == "clip":
        x = jnp.clip(x, -1.0, 1.0)
    else:
        x = x - jnp.max(x, axis=1, keepdims=True)
        ex = jnp.exp(x)
        x = ex / jnp.sum(ex, axis=1, keepdims=True)
    x_ref[...] = x
    xs = dis * x
    zpad = jnp.zeros((BM, CP - CH), jnp.float32)
    xsa_ref[...] = jnp.concatenate([xs[:, :CH], zpad], axis=1)
    xsb_ref[...] = jnp.concatenate([xs[:, CH:], zpad], axis=1)


def _mid_tc(x10_ref, y_soft_ref, y_true_ref, cnt_ref, dis_ref, sigma_ref,
            y_ref, ysa_ref, ysb_ref):
    x10 = x10_ref[...]
    sigma = sigma_ref[...][0, 0] * (1.0 / 50000.0)
    denom = jnp.sum(jnp.abs(x10), axis=1, keepdims=True)
    bad = (denom <= 1e-12) | ((sigma / jnp.maximum(denom, 1e-12)) > 1000.0)
    scale = jnp.where(bad, 1.0, sigma / jnp.where(bad, 1.0, denom))
    y_corr = y_soft_ref[...] + scale * x10
    yt = y_true_ref[...]
    oh = (lax.broadcasted_iota(jnp.int32, (BM, C), 1) == yt).astype(jnp.float32)
    y = jnp.where(cnt_ref[...] > 0, oh, y_corr)
    y_ref[...] = y
    ys = dis_ref[...] * y
    zpad = jnp.zeros((BM, CP - CH), jnp.float32)
    ysa_ref[...] = jnp.concatenate([ys[:, :CH], zpad], axis=1)
    ysb_ref[...] = jnp.concatenate([ys[:, CH:], zpad], axis=1)


def _row_spec(w):
    return pl.BlockSpec((BM, w), lambda i: (i, 0))


def _one_spec():
    return pl.BlockSpec((1, 1), lambda i: (0, 0))


_prep_call = pl.pallas_call(
    _prep_tc,
    grid=(GRID,),
    in_specs=[_row_spec(1), _row_spec(C), _row_spec(1), _row_spec(1)],
    out_specs=[_row_spec(1), _row_spec(C), _row_spec(CP), _row_spec(CP),
               _one_spec()],
    out_shape=[jax.ShapeDtypeStruct((N, 1), jnp.float32),
               jax.ShapeDtypeStruct((N, C), jnp.float32),
               jax.ShapeDtypeStruct((N, CP), jnp.float32),
               jax.ShapeDtypeStruct((N, CP), jnp.float32),
               jax.ShapeDtypeStruct((1, 1), jnp.float32)],
)


def _post_call(alpha, mode):
    return pl.pallas_call(
        partial(_post_tc, alpha=alpha, mode=mode),
        grid=(GRID,),
        in_specs=[_row_spec(CP), _row_spec(CP), _row_spec(C), _row_spec(1)],
        out_specs=[_row_spec(C), _row_spec(CP), _row_spec(CP)],
        out_shape=[jax.ShapeDtypeStruct((N, C), jnp.float32),
                   jax.ShapeDtypeStruct((N, CP), jnp.float32),
                   jax.ShapeDtypeStruct((N, CP), jnp.float32)],
    )


_mid_call = pl.pallas_call(
    _mid_tc,
    grid=(GRID,),
    in_specs=[_row_spec(C), _row_spec(C), _row_spec(1), _row_spec(1),
              _row_spec(1), _one_spec()],
    out_specs=[_row_spec(C), _row_spec(CP), _row_spec(CP)],
    out_shape=[jax.ShapeDtypeStruct((N, C), jnp.float32),
               jax.ShapeDtypeStruct((N, CP), jnp.float32),
               jax.ShapeDtypeStruct((N, CP), jnp.float32)],
)

_post_c = _post_call(0.9, "clip")
_post_s = _post_call(0.8, "softmax")


def kernel(y_true, y_soft, spread_idx, eval_idx, test_idx, edge_index):
    src = edge_index[0].astype(jnp.int32)
    dst = edge_index[1].astype(jnp.int32)
    spread = spread_idx.astype(jnp.int32)

    slot = jnp.arange(ETOT, dtype=jnp.int32)
    w = slot // SLOTS
    r = slot - w * SLOTS
    e = jnp.minimum(w * (E // 16) + r, E - 1)
    valid = r < (E // 16)
    sv = src[e]
    dv = dst[e]
    dumpv = HALF + (slot & (NDUMP - 1))
    padsrc = slot % N
    srcp = jnp.where(valid, sv, padsrc)
    dstl = jnp.concatenate([
        jnp.where(valid & (dv < HALF), dv, dumpv),
        jnp.where(valid & (dv >= HALF), dv - HALF, dumpv)])

    s_iota = jnp.arange(SPAD, dtype=jnp.int32)
    sdump = HALF + (s_iota & (NDUMP - 1))
    spad = jnp.concatenate([spread, jnp.full((SPAD - 50000,), -1, jnp.int32)])
    sprl = jnp.concatenate([
        jnp.where((spad >= 0) & (spad < HALF), spad, sdump),
        jnp.where(spad >= HALF, spad - HALF, sdump)])

    deg, cnt = _hist_call(dstl, sprl)

    yt2 = y_true.astype(jnp.int32).reshape(N, 1)
    deg2 = deg.reshape(N, 1)
    cnt2 = cnt.reshape(N, 1)
    dis, e0, xsa, xsb, sigma = _prep_call(yt2, y_soft, deg2, cnt2)

    x = e0
    for _ in range(10):
        agga, aggb = _agg_call(xsa, xsb, srcp, dstl)
        x, xsa, xsb = _post_c(agga, aggb, e0, dis)

    y, xsa, xsb = _mid_call(x, y_soft, yt2, cnt2, dis, sigma)

    x = y
    for _ in range(10):
        agga, aggb = _agg_call(xsa, xsb, srcp, dstl)
        x, xsa, xsb = _post_s(agga, aggb, y, dis)

    return x


# speedup vs baseline: 7.6929x; 1.0001x over previous
from functools import partial

import jax
import jax.numpy as jnp
from jax import lax
from jax.experimental import pallas as pl
from jax.experimental.pallas import tpu as pltpu, tpu_sc as plsc

N = 100000
E = 1600000
C = 40
CH = C // 2
CP = 24
HALF = N // 2
NDUMP = 16
ACC_ROWS = HALF + NDUMP
B = 128
K = 8
IT = 100
SLOTS = IT * K * B
ETOT = 16 * SLOTS
DRA = 3128
DRB = 50000 - 15 * DRA
SPAD = 51200
SSPW = SPAD // 16

_mesh = plsc.VectorSubcoreMesh(core_axis_name="c", subcore_axis_name="s")


def _z16():
    return jnp.zeros((16,), jnp.float32)


def _agg_body(xsa_hbm, xsb_hbm, srcp_hbm, dstl_hbm, agga_hbm, aggb_hbm,
              acc, dbuf, *scr):
    sidx = scr[0:8]
    didx = scr[8:16]
    rows = scr[16:24]
    gsem, isem_s, isem_d = scr[24:27]
    c = lax.axis_index("c")
    s = lax.axis_index("s")

    def zero_dbuf():
        def zr(i, _):
            z = _z16()
            dbuf[i, pl.ds(0, 16)] = z
            dbuf[i, pl.ds(8, 16)] = z
            return 0
        lax.fori_loop(0, 1024, zr, 0)

    def zero_acc():
        zbase = s * (ACC_ROWS // 16)

        def zc(k, _):
            pltpu.sync_copy(dbuf, acc.at[pl.ds(zbase + k * 1024, 1024)])
            return 0
        lax.fori_loop(0, 3, zc, 0)
        pltpu.sync_copy(dbuf.at[pl.ds(0, 54)],
                        acc.at[pl.ds(zbase + 3072, 54)])

    sbase = s * SLOTS
    dbase0 = s * SLOTS
    zero_dbuf()

    def run_pass(xs_hbm, agg_hbm):
        zero_acc()
        plsc.subcore_barrier()

        def i_issue(j, b):
            off = sbase + j * B
            doff = c * ETOT + dbase0 + j * B
            pltpu.async_copy(srcp_hbm.at[pl.ds(off, B)], sidx[b],
                             isem_s.at[b])
            pltpu.async_copy(dstl_hbm.at[pl.ds(doff, B)], didx[b],
                             isem_d.at[b])

        def i_wait(b):
            pltpu.make_async_copy(srcp_hbm.at[pl.ds(sbase, B)], sidx[b],
                                  isem_s.at[b]).wait()
            pltpu.make_async_copy(dstl_hbm.at[pl.ds(sbase, B)], didx[b],
                                  isem_d.at[b]).wait()

        def g_issue(b):
            pltpu.async_copy(xs_hbm.at[sidx[b]], rows[b], gsem.at[b])

        def g_wait(b):
            pltpu.make_async_copy(xs_hbm.at[sidx[b]], rows[b],
                                  gsem.at[b]).wait()

        for b in range(8):
            i_issue(b, b)

        def body(i, _):
            for h in range(2):
                sl = range(4 * h, 4 * h + 4)
                for b in sl:
                    i_wait(b)
                    g_issue(b)
                for b in sl:
                    g_wait(b)
                    pltpu.sync_copy(rows[b], acc.at[didx[b]], add=True)
                    i_issue(jnp.minimum(8 * i + 8 + b, SLOTS // B - 1), b)
            return 0
        lax.fori_loop(0, IT, body, 0)
        for b in range(8):
            i_wait(b)

        plsc.subcore_barrier()

        def dchunk(base, sz):
            pltpu.sync_copy(acc.at[pl.ds(base, sz)], dbuf.at[pl.ds(0, sz)])
            pltpu.sync_copy(dbuf.at[pl.ds(0, sz)],
                            agg_hbm.at[pl.ds(c * HALF + base, sz)])

        @pl.when(s < 15)
        def _():
            def dc(k, _):
                base = s * DRA + k * 1024
                pltpu.sync_copy(acc.at[pl.ds(base, 1024)], dbuf)
                pltpu.sync_copy(dbuf,
                                agg_hbm.at[pl.ds(c * HALF + base, 1024)])
                return 0
            lax.fori_loop(0, 3, dc, 0)
            dchunk(s * DRA + 3072, DRA - 3072)

        @pl.when(s == 15)
        def _():
            def dc(k, _):
                base = 15 * DRA + k * 1024
                pltpu.sync_copy(acc.at[pl.ds(base, 1024)], dbuf)
                pltpu.sync_copy(dbuf,
                                agg_hbm.at[pl.ds(c * HALF + base, 1024)])
                return 0
            lax.fori_loop(0, 3, dc, 0)
            dchunk(15 * DRA + 3072, DRB - 3072)
        zero_dbuf()
        plsc.subcore_barrier()

    run_pass(xsa_hbm, agga_hbm)
    run_pass(xsb_hbm, aggb_hbm)


_agg_call = pl.kernel(
    _agg_body,
    out_type=(jax.ShapeDtypeStruct((N, CP), jnp.float32),
              jax.ShapeDtypeStruct((N, CP), jnp.float32)),
    mesh=_mesh,
    scratch_types=(
        [pltpu.VMEM_SHARED((ACC_ROWS, CP), jnp.float32),
         pltpu.VMEM((1024, CP), jnp.float32)]
        + [pltpu.VMEM((B,), jnp.int32) for _ in range(16)]
        + [pltpu.VMEM((B, CP), jnp.float32) for _ in range(8)]
        + [pltpu.SemaphoreType.DMA((8,)) for _ in range(3)]
    ),
    compiler_params=pltpu.CompilerParams(use_tc_tiling_on_sc=False),
)


def _hist_body(dstl_hbm, sprl_hbm, deg_hbm, cnt_hbm,
               acc_d, acc_c, ones, zrow, idxb):
    c = lax.axis_index("c")
    s = lax.axis_index("s")

    def fill(i, _):
        ones[pl.ds(i * 16, 16)] = _z16() + 1.0
        zrow[pl.ds(i * 16, 16)] = _z16()
        return 0
    lax.fori_loop(0, 8, fill, 0)

    def zseg(i, _):
        zrow[pl.ds(i * 16 + 128, 16)] = _z16()
        return 0
    lax.fori_loop(0, (3200 - 128) // 16, zseg, 0)

    ZRA, ZRB = 3128, ACC_ROWS - 15 * 3128

    @pl.when(s < 15)
    def _():
        pltpu.sync_copy(zrow.at[pl.ds(0, ZRA)], acc_d.at[pl.ds(s * ZRA, ZRA)])
        pltpu.sync_copy(zrow.at[pl.ds(0, ZRA)], acc_c.at[pl.ds(s * ZRA, ZRA)])

    @pl.when(s == 15)
    def _():
        pltpu.sync_copy(zrow.at[pl.ds(0, ZRB)], acc_d.at[pl.ds(15 * ZRA, ZRB)])
        pltpu.sync_copy(zrow.at[pl.ds(0, ZRB)], acc_c.at[pl.ds(15 * ZRA, ZRB)])
    plsc.subcore_barrier()

    ebase = s * SLOTS

    def dbody(j, _):
        pltpu.sync_copy(dstl_hbm.at[pl.ds(c * ETOT + ebase + j * B, B)], idxb)
        pltpu.sync_copy(ones.at[pl.ds(0, B)], acc_d.at[idxb], add=True)
        return 0
    lax.fori_loop(0, SLOTS // B, dbody, 0)

    sbase = s * SSPW

    def sbody(j, _):
        pltpu.sync_copy(sprl_hbm.at[pl.ds(c * SPAD + sbase + j * B, B)], idxb)
        pltpu.sync_copy(ones.at[pl.ds(0, B)], acc_c.at[idxb], add=True)
        return 0
    lax.fori_loop(0, SSPW // B, sbody, 0)

    plsc.subcore_barrier()

    @pl.when(s < 15)
    def _():
        pltpu.sync_copy(acc_d.at[pl.ds(s * DRA, DRA)], zrow.at[pl.ds(0, DRA)])
        pltpu.sync_copy(zrow.at[pl.ds(0, DRA)],
                        deg_hbm.at[pl.ds(c * HALF + s * DRA, DRA)])
        pltpu.sync_copy(acc_c.at[pl.ds(s * DRA, DRA)], zrow.at[pl.ds(0, DRA)])
        pltpu.sync_copy(zrow.at[pl.ds(0, DRA)],
                        cnt_hbm.at[pl.ds(c * HALF + s * DRA, DRA)])

    @pl.when(s == 15)
    def _():
        pltpu.sync_copy(acc_d.at[pl.ds(15 * DRA, DRB)], zrow.at[pl.ds(0, DRB)])
        pltpu.sync_copy(zrow.at[pl.ds(0, DRB)],
                        deg_hbm.at[pl.ds(c * HALF + 15 * DRA, DRB)])
        pltpu.sync_copy(acc_c.at[pl.ds(15 * DRA, DRB)], zrow.at[pl.ds(0, DRB)])
        pltpu.sync_copy(zrow.at[pl.ds(0, DRB)],
                        cnt_hbm.at[pl.ds(c * HALF + 15 * DRA, DRB)])


_hist_call = pl.kernel(
    _hist_body,
    out_type=(jax.ShapeDtypeStruct((N,), jnp.float32),
              jax.ShapeDtypeStruct((N,), jnp.float32)),
    mesh=_mesh,
    scratch_types=[
        pltpu.VMEM_SHARED((ACC_ROWS,), jnp.float32),
        pltpu.VMEM_SHARED((ACC_ROWS,), jnp.float32),
        pltpu.VMEM((B,), jnp.float32),
        pltpu.VMEM((3200,), jnp.float32),
        pltpu.VMEM((B,), jnp.int32),
    ],
    compiler_params=pltpu.CompilerParams(use_tc_tiling_on_sc=False),
)


BM = 1000
GRID = N // BM


def _prep_tc(y_true_ref, y_soft_ref, deg_ref, cnt_ref,
             dis_ref, e0_ref, e0sa_ref, e0sb_ref, sigma_ref):
    i = pl.program_id(0)
    deg = deg_ref[...]
    cnt = cnt_ref[...]
    dis = jnp.where(deg > 0, lax.rsqrt(jnp.maximum(deg, 1.0)), 0.0)
    dis_ref[...] = dis
    yt = y_true_ref[...]
    oh = (lax.broadcasted_iota(jnp.int32, (BM, C), 1) == yt).astype(jnp.float32)
    e0 = jnp.where(cnt > 0, oh - y_soft_ref[...], 0.0)
    e0_ref[...] = e0
    e0s = dis * e0
    zpad = jnp.zeros((BM, CP - CH), jnp.float32)
    e0sa_ref[...] = jnp.concatenate([e0s[:, :CH], zpad], axis=1)
    e0sb_ref[...] = jnp.concatenate([e0s[:, CH:], zpad], axis=1)
    part = jnp.sum(cnt * jnp.sum(jnp.abs(e0), axis=1, keepdims=True))

    @pl.when(i == 0)
    def _():
        sigma_ref[...] = jnp.zeros_like(sigma_ref)
    sigma_ref[...] = sigma_ref[...] + part


def _post_tc(agga_ref, aggb_ref, y0_ref, dis_ref, x_ref, xsa_ref, xsb_ref,
             *, alpha, mode):
    dis = dis_ref[...]
    agg = jnp.concatenate([agga_ref[...][:, :CH], aggb_ref[...][:, :CH]],
                          axis=1)
    x = alpha * dis * agg + (1.0 - alpha) * y0_ref[...]
    if mode == "clip":
        x = jnp.clip(x, -1.0, 1.0)
    else:
        x = x - jnp.max(x, axis=1, keepdims=True)
        ex = jnp.exp(x)
        x = ex / jnp.sum(ex, axis=1, keepdims=True)
    x_ref[...] = x
    xs = dis * x
    zpad = jnp.zeros((BM, CP - CH), jnp.float32)
    xsa_ref[...] = jnp.concatenate([xs[:, :CH], zpad], axis=1)
    xsb_ref[...] = jnp.concatenate([xs[:, CH:], zpad], axis=1)


def _mid_tc(x10_ref, y_soft_ref, y_true_ref, cnt_ref, dis_ref, sigma_ref,
            y_ref, ysa_ref, ysb_ref):
    x10 = x10_ref[...]
    sigma = sigma_ref[...][0, 0] * (1.0 / 50000.0)
    denom = jnp.sum(jnp.abs(x10), axis=1, keepdims=True)
    bad = (denom <= 1e-12) | ((sigma / jnp.maximum(denom, 1e-12)) > 1000.0)
    scale = jnp.where(bad, 1.0, sigma / jnp.where(bad, 1.0, denom))
    y_corr = y_soft_ref[...] + scale * x10
    yt = y_true_ref[...]
    oh = (lax.broadcasted_iota(jnp.int32, (BM, C), 1) == yt).astype(jnp.float32)
    y = jnp.where(cnt_ref[...] > 0, oh, y_corr)
    y_ref[...] = y
    ys = dis_ref[...] * y
    zpad = jnp.zeros((BM, CP - CH), jnp.float32)
    ysa_ref[...] = jnp.concatenate([ys[:, :CH], zpad], axis=1)
    ysb_ref[...] = jnp.concatenate([ys[:, CH:], zpad], axis=1)


def _row_spec(w):
    return pl.BlockSpec((BM, w), lambda i: (i, 0))


def _one_spec():
    return pl.BlockSpec((1, 1), lambda i: (0, 0))


_prep_call = pl.pallas_call(
    _prep_tc,
    grid=(GRID,),
    in_specs=[_row_spec(1), _row_spec(C), _row_spec(1), _row_spec(1)],
    out_specs=[_row_spec(1), _row_spec(C), _row_spec(CP), _row_spec(CP),
               _one_spec()],
    out_shape=[jax.ShapeDtypeStruct((N, 1), jnp.float32),
               jax.ShapeDtypeStruct((N, C), jnp.float32),
               jax.ShapeDtypeStruct((N, CP), jnp.float32),
               jax.ShapeDtypeStruct((N, CP), jnp.float32),
               jax.ShapeDtypeStruct((1, 1), jnp.float32)],
)


def _post_call(alpha, mode):
    return pl.pallas_call(
        partial(_post_tc, alpha=alpha, mode=mode),
        grid=(GRID,),
        in_specs=[_row_spec(CP), _row_spec(CP), _row_spec(C), _row_spec(1)],
        out_specs=[_row_spec(C), _row_spec(CP), _row_spec(CP)],
        out_shape=[jax.ShapeDtypeStruct((N, C), jnp.float32),
                   jax.ShapeDtypeStruct((N, CP), jnp.float32),
                   jax.ShapeDtypeStruct((N, CP), jnp.float32)],
    )


_mid_call = pl.pallas_call(
    _mid_tc,
    grid=(GRID,),
    in_specs=[_row_spec(C), _row_spec(C), _row_spec(1), _row_spec(1),
              _row_spec(1), _one_spec()],
    out_specs=[_row_spec(C), _row_spec(CP), _row_spec(CP)],
    out_shape=[jax.ShapeDtypeStruct((N, C), jnp.float32),
               jax.ShapeDtypeStruct((N, CP), jnp.float32),
               jax.ShapeDtypeStruct((N, CP), jnp.float32)],
)

_post_c = _post_call(0.9, "clip")
_post_s = _post_call(0.8, "softmax")


def kernel(y_true, y_soft, spread_idx, eval_idx, test_idx, edge_index):
    src = edge_index[0].astype(jnp.int32)
    dst = edge_index[1].astype(jnp.int32)
    spread = spread_idx.astype(jnp.int32)

    slot = jnp.arange(ETOT, dtype=jnp.int32)
    w = slot // SLOTS
    r = slot - w * SLOTS
    e = jnp.minimum(w * (E // 16) + r, E - 1)
    valid = r < (E // 16)
    sv = src[e]
    dv = dst[e]
    dumpv = HALF + (slot & (NDUMP - 1))
    padsrc = slot % N
    srcp = jnp.where(valid, sv, padsrc)
    dstl = jnp.concatenate([
        jnp.where(valid & (dv < HALF), dv, dumpv),
        jnp.where(valid & (dv >= HALF), dv - HALF, dumpv)])

    s_iota = jnp.arange(SPAD, dtype=jnp.int32)
    sdump = HALF + (s_iota & (NDUMP - 1))
    spad = jnp.concatenate([spread, jnp.full((SPAD - 50000,), -1, jnp.int32)])
    sprl = jnp.concatenate([
        jnp.where((spad >= 0) & (spad < HALF), spad, sdump),
        jnp.where(spad >= HALF, spad - HALF, sdump)])

    deg, cnt = _hist_call(dstl, sprl)

    yt2 = y_true.astype(jnp.int32).reshape(N, 1)
    deg2 = deg.reshape(N, 1)
    cnt2 = cnt.reshape(N, 1)
    dis, e0, xsa, xsb, sigma = _prep_call(yt2, y_soft, deg2, cnt2)

    x = e0
    for _ in range(10):
        agga, aggb = _agg_call(xsa, xsb, srcp, dstl)
        x, xsa, xsb = _post_c(agga, aggb, e0, dis)

    y, xsa, xsb = _mid_call(x, y_soft, yt2, cnt2, dis, sigma)

    x = y
    for _ in range(10):
        agga, aggb = _agg_call(xsa, xsb, srcp, dstl)
        x, xsa, xsb = _post_s(agga, aggb, y, dis)

    return x
